# baseline (speedup 1.0000x reference)
import functools

import jax
import jax.numpy as jnp
from jax import lax
from jax.experimental import pallas as pl
from jax.experimental.pallas import tpu as pltpu
from jax.experimental.pallas import tpu_sc as plsc

N_NODES = 10000
C_DIM = 128
K_OFF = 27
E_EDGES = 12000

NUM_CORES = 2
NUM_SUBCORES = 16
NUM_TILES = NUM_CORES * NUM_SUBCORES

BATCH = 128
EDGES_TOTAL = K_OFF * E_EDGES
EPT = 10240
NBATCH = EPT // BATCH
N_PAD = 10240
ROWS_PER_TILE = N_PAD // NUM_SUBCORES

BM = 2000
NB = N_NODES // BM
BM3 = 1024
NBLK3 = N_PAD // BM3
EPS = 1e-5


def _mm_body(x_ref, w_ref, y_ref):
    y_ref[...] = jnp.dot(x_ref[...], w_ref[0],
                         preferred_element_type=jnp.float32)


def _per_offset_matmul(x, w):
    return pl.pallas_call(
        _mm_body,
        grid=(NB, K_OFF),
        in_specs=[
            pl.BlockSpec((BM, C_DIM), lambda i, k: (i, 0)),
            pl.BlockSpec((1, C_DIM, C_DIM), lambda i, k: (k, 0, 0)),
        ],
        out_specs=pl.BlockSpec((BM, C_DIM), lambda i, k: (k * NB + i, 0)),
        out_shape=jax.ShapeDtypeStruct((K_OFF * N_NODES, C_DIM), jnp.float32),
    )(x, w)


def _merge_stats_body(p_ref, a_ref, stats_ref):
    i = pl.program_id(0)
    a = p_ref[0] + p_ref[1]
    a_ref[...] = a
    rows = lax.broadcasted_iota(jnp.int32, (BM3, C_DIM), 0) + i * BM3
    am = jnp.where(rows < N_NODES, a, 0.0)
    s = jnp.sum(am, axis=0, keepdims=True)
    sq = jnp.sum(am * am, axis=0, keepdims=True)
    upd = jnp.concatenate([s, sq, jnp.zeros((6, C_DIM), jnp.float32)], axis=0)

    @pl.when(i == 0)
    def _():
        stats_ref[...] = jnp.zeros((8, C_DIM), jnp.float32)

    stats_ref[...] += upd


def _merge_and_stats(partials):
    return pl.pallas_call(
        _merge_stats_body,
        grid=(NBLK3,),
        in_specs=[pl.BlockSpec((2, BM3, C_DIM), lambda i: (0, i, 0))],
        out_specs=[
            pl.BlockSpec((BM3, C_DIM), lambda i: (i, 0)),
            pl.BlockSpec((8, C_DIM), lambda i: (0, 0)),
        ],
        out_shape=[
            jax.ShapeDtypeStruct((N_PAD, C_DIM), jnp.float32),
            jax.ShapeDtypeStruct((8, C_DIM), jnp.float32),
        ],
    )(partials)


def _bn_mm_body(a_ref, stats_ref, gb_ref, w_ref, y_ref):
    st = stats_ref[...]
    gb = gb_ref[...]
    mean = st[0] * (1.0 / N_NODES)
    var = st[1] * (1.0 / N_NODES) - mean * mean
    inv = lax.rsqrt(var + EPS)
    scale = inv * gb[0]
    shift = gb[1] - mean * scale
    h = jnp.maximum(a_ref[...] * scale + shift, 0.0)
    y_ref[...] = jnp.dot(h, w_ref[0], preferred_element_type=jnp.float32)


def _bn_relu_matmul(a, stats, gb, w):
    return pl.pallas_call(
        _bn_mm_body,
        grid=(NB, K_OFF),
        in_specs=[
            pl.BlockSpec((BM, C_DIM), lambda i, k: (i, 0)),
            pl.BlockSpec((8, C_DIM), lambda i, k: (0, 0)),
            pl.BlockSpec((8, C_DIM), lambda i, k: (0, 0)),
            pl.BlockSpec((1, C_DIM, C_DIM), lambda i, k: (k, 0, 0)),
        ],
        out_specs=pl.BlockSpec((BM, C_DIM), lambda i, k: (k * NB + i, 0)),
        out_shape=jax.ShapeDtypeStruct((K_OFF * N_NODES, C_DIM), jnp.float32),
    )(a, stats, gb, w)


def _final_body(p_ref, x_ref, o_ref):
    o_ref[...] = jnp.maximum(p_ref[0] + p_ref[1] + x_ref[...], 0.0)


def _residual_relu(partials, x):
    return pl.pallas_call(
        _final_body,
        grid=(NB,),
        in_specs=[
            pl.BlockSpec((2, BM, C_DIM), lambda i: (0, i, 0)),
            pl.BlockSpec((BM, C_DIM), lambda i: (i, 0)),
        ],
        out_specs=pl.BlockSpec((BM, C_DIM), lambda i: (i, 0)),
        out_shape=jax.ShapeDtypeStruct((N_NODES, C_DIM), jnp.float32),
    )(partials, x)


NBUF = 1
GROUP = 2
GB = GROUP * BATCH
CHUNK = 16
NCHUNK = NBATCH // CHUNK


def _sc_scatter_body(y_hbm, inidx_hbm, outidx_hbm, out_hbm,
                     in_v, out_v, r0,
                     acc_sh, g0):
    rows = (r0,)
    gsem = (g0,)
    cid = lax.axis_index("c")
    sid = lax.axis_index("s")
    wid = cid * NUM_SUBCORES + sid

    zero = jnp.zeros((16,), jnp.float32)

    def zrow(r, carry):
        for c8 in range(C_DIM // 16):
            r0[r, pl.ds(c8 * 16, 16)] = zero
        return carry

    lax.fori_loop(0, BATCH, zrow, 0)
    for b in range(ROWS_PER_TILE // BATCH):
        pltpu.sync_copy(r0.at[pl.ds(0, BATCH)],
                        acc_sh.at[pl.ds(sid * ROWS_PER_TILE + b * BATCH,
                                        BATCH)])

    plsc.subcore_barrier()

    def chunk_body(cc, carry):
        pltpu.sync_copy(inidx_hbm.at[wid, cc], in_v)
        pltpu.sync_copy(outidx_hbm.at[wid, cc], out_v)

        def body(j, c2):
            pltpu.async_copy(y_hbm.at[in_v.at[pl.ds(j * GB, GB)]],
                             rows[0], gsem[0]).wait()
            pltpu.sync_copy(rows[0],
                            acc_sh.at[out_v.at[pl.ds(j * GB, GB)]],
                            add=True)
            return c2

        lax.fori_loop(0, CHUNK // GROUP, body, 0)
        return carry

    lax.fori_loop(0, NCHUNK, chunk_body, 0)

    plsc.subcore_barrier()

    pltpu.sync_copy(acc_sh.at[pl.ds(sid * ROWS_PER_TILE, ROWS_PER_TILE)],
                    out_hbm.at[cid, pl.ds(sid * ROWS_PER_TILE,
                                          ROWS_PER_TILE)])


@functools.lru_cache(maxsize=None)
def _build_sc_scatter():
    return pl.kernel(
        _sc_scatter_body,
        out_type=jax.ShapeDtypeStruct((NUM_CORES, N_PAD, C_DIM), jnp.float32),
        mesh=plsc.VectorSubcoreMesh(core_axis_name="c", subcore_axis_name="s",
                                    num_cores=NUM_CORES,
                                    num_subcores=NUM_SUBCORES),
        scratch_types=(
            [pltpu.VMEM((CHUNK * BATCH,), jnp.int32)] * 2
            + [pltpu.VMEM((GB, C_DIM), jnp.float32)] * NBUF
            + [pltpu.VMEM_SHARED((N_PAD, C_DIM), jnp.float32)]
            + [pltpu.SemaphoreType.DMA] * NBUF
        ),
    )


def kernel(x, in_idx, out_idx, W1, W2, gamma, beta):
    koffs = (jnp.arange(K_OFF, dtype=jnp.int32) * N_NODES)[:, None]
    in_flat = (in_idx.astype(jnp.int32) + koffs).reshape(-1)
    out_flat = out_idx.astype(jnp.int32).reshape(-1)
    pad = NUM_TILES * EPT - EDGES_TOTAL
    in_flat = jnp.concatenate([in_flat, jnp.zeros((pad,), jnp.int32)])
    out_flat = jnp.concatenate(
        [out_flat, jnp.full((pad,), N_NODES, jnp.int32)])
    in3 = in_flat.reshape(NUM_TILES, NCHUNK, CHUNK * BATCH)
    out3 = out_flat.reshape(NUM_TILES, NCHUNK, CHUNK * BATCH)

    gb = jnp.zeros((8, C_DIM), jnp.float32).at[0].set(gamma).at[1].set(beta)

    sc_scatter = _build_sc_scatter()
    y1 = _per_offset_matmul(x, W1)
    p1 = sc_scatter(y1, in3, out3)
    a, stats = _merge_and_stats(p1)
    y2 = _bn_relu_matmul(a, stats, gb, W2)
    p2 = sc_scatter(y2, in3, out3)
    return _residual_relu(p2, x)


# speedup vs baseline: 1.7565x; 1.7565x over previous
import functools

import jax
import jax.numpy as jnp
from jax import lax
from jax.experimental import pallas as pl
from jax.experimental.pallas import tpu as pltpu
from jax.experimental.pallas import tpu_sc as plsc

N_NODES = 10000
C_DIM = 128
K_OFF = 27
E_EDGES = 12000

NUM_CORES = 2
NUM_SUBCORES = 16
NUM_TILES = NUM_CORES * NUM_SUBCORES

BATCH = 128
EDGES_TOTAL = K_OFF * E_EDGES
EPT = 10240
NBATCH = EPT // BATCH
N_PAD = 10240
ROWS_PER_TILE = N_PAD // NUM_SUBCORES

BM = 2000
NB = N_NODES // BM
BM3 = 1024
NBLK3 = N_PAD // BM3
EPS = 1e-5


def _mm_body(x_ref, w_ref, y_ref):
    y_ref[...] = jnp.dot(x_ref[...], w_ref[0],
                         preferred_element_type=jnp.float32)


def _per_offset_matmul(x, w):
    return pl.pallas_call(
        _mm_body,
        grid=(NB, K_OFF),
        in_specs=[
            pl.BlockSpec((BM, C_DIM), lambda i, k: (i, 0)),
            pl.BlockSpec((1, C_DIM, C_DIM), lambda i, k: (k, 0, 0)),
        ],
        out_specs=pl.BlockSpec((BM, C_DIM), lambda i, k: (k * NB + i, 0)),
        out_shape=jax.ShapeDtypeStruct((K_OFF * N_NODES, C_DIM), jnp.float32),
    )(x, w)


def _merge_stats_body(p_ref, a_ref, stats_ref):
    i = pl.program_id(0)
    a = p_ref[0] + p_ref[1]
    a_ref[...] = a
    rows = lax.broadcasted_iota(jnp.int32, (BM3, C_DIM), 0) + i * BM3
    am = jnp.where(rows < N_NODES, a, 0.0)
    s = jnp.sum(am, axis=0, keepdims=True)
    sq = jnp.sum(am * am, axis=0, keepdims=True)
    upd = jnp.concatenate([s, sq, jnp.zeros((6, C_DIM), jnp.float32)], axis=0)

    @pl.when(i == 0)
    def _():
        stats_ref[...] = jnp.zeros((8, C_DIM), jnp.float32)

    stats_ref[...] += upd


def _merge_and_stats(partials):
    return pl.pallas_call(
        _merge_stats_body,
        grid=(NBLK3,),
        in_specs=[pl.BlockSpec((2, BM3, C_DIM), lambda i: (0, i, 0))],
        out_specs=[
            pl.BlockSpec((BM3, C_DIM), lambda i: (i, 0)),
            pl.BlockSpec((8, C_DIM), lambda i: (0, 0)),
        ],
        out_shape=[
            jax.ShapeDtypeStruct((N_PAD, C_DIM), jnp.float32),
            jax.ShapeDtypeStruct((8, C_DIM), jnp.float32),
        ],
    )(partials)


def _bn_mm_body(a_ref, stats_ref, gb_ref, w_ref, y_ref):
    st = stats_ref[...]
    gb = gb_ref[...]
    mean = st[0] * (1.0 / N_NODES)
    var = st[1] * (1.0 / N_NODES) - mean * mean
    inv = lax.rsqrt(var + EPS)
    scale = inv * gb[0]
    shift = gb[1] - mean * scale
    h = jnp.maximum(a_ref[...] * scale + shift, 0.0)
    y_ref[...] = jnp.dot(h, w_ref[0], preferred_element_type=jnp.float32)


def _bn_relu_matmul(a, stats, gb, w):
    return pl.pallas_call(
        _bn_mm_body,
        grid=(NB, K_OFF),
        in_specs=[
            pl.BlockSpec((BM, C_DIM), lambda i, k: (i, 0)),
            pl.BlockSpec((8, C_DIM), lambda i, k: (0, 0)),
            pl.BlockSpec((8, C_DIM), lambda i, k: (0, 0)),
            pl.BlockSpec((1, C_DIM, C_DIM), lambda i, k: (k, 0, 0)),
        ],
        out_specs=pl.BlockSpec((BM, C_DIM), lambda i, k: (k * NB + i, 0)),
        out_shape=jax.ShapeDtypeStruct((K_OFF * N_NODES, C_DIM), jnp.float32),
    )(a, stats, gb, w)


def _final_body(p_ref, x_ref, o_ref):
    o_ref[...] = jnp.maximum(p_ref[0] + p_ref[1] + x_ref[...], 0.0)


def _residual_relu(partials, x):
    return pl.pallas_call(
        _final_body,
        grid=(NB,),
        in_specs=[
            pl.BlockSpec((2, BM, C_DIM), lambda i: (0, i, 0)),
            pl.BlockSpec((BM, C_DIM), lambda i: (i, 0)),
        ],
        out_specs=pl.BlockSpec((BM, C_DIM), lambda i: (i, 0)),
        out_shape=jax.ShapeDtypeStruct((N_NODES, C_DIM), jnp.float32),
    )(partials, x)


NBUF = 2
CHUNK = 16
NCHUNK = NBATCH // CHUNK


def _sc_scatter_body(y_hbm, inidx_hbm, outidx_hbm, out_hbm,
                     in_v, out_v, r0, r1,
                     acc_sh, g0, g1, s0, s1):
    rows = (r0, r1)
    gsem = (g0, g1)
    ssem = (s0, s1)
    cid = lax.axis_index("c")
    sid = lax.axis_index("s")
    wid = cid * NUM_SUBCORES + sid

    zero = jnp.zeros((16,), jnp.float32)

    def zrow(r, carry):
        for c8 in range(C_DIM // 16):
            r0[r, pl.ds(c8 * 16, 16)] = zero
        return carry

    lax.fori_loop(0, BATCH, zrow, 0)
    for b in range(ROWS_PER_TILE // BATCH):
        pltpu.sync_copy(r0,
                        acc_sh.at[pl.ds(sid * ROWS_PER_TILE + b * BATCH,
                                        BATCH)])

    plsc.subcore_barrier()

    def chunk_body(cc, carry):
        pltpu.sync_copy(inidx_hbm.at[wid, cc], in_v)
        pltpu.sync_copy(outidx_hbm.at[wid, cc], out_v)
        for b in range(NBUF):
            pltpu.async_copy(y_hbm.at[in_v.at[b]], rows[b], gsem[b])

        def body(jj, c2):
            for b in range(NBUF):
                j = jj * NBUF + b
                pltpu.make_async_copy(y_hbm.at[in_v.at[j]], rows[b],
                                      gsem[b]).wait()
                pltpu.sync_copy(rows[b], acc_sh.at[out_v.at[j]], add=True)

                @pl.when(j + NBUF < CHUNK)
                def _():
                    pltpu.async_copy(y_hbm.at[in_v.at[j + NBUF]], rows[b],
                                     gsem[b])

            return c2

        lax.fori_loop(0, CHUNK // NBUF, body, 0)
        return carry

    lax.fori_loop(0, NCHUNK, chunk_body, 0)

    plsc.subcore_barrier()

    pltpu.sync_copy(acc_sh.at[pl.ds(sid * ROWS_PER_TILE, ROWS_PER_TILE)],
                    out_hbm.at[cid, pl.ds(sid * ROWS_PER_TILE,
                                          ROWS_PER_TILE)])


@functools.lru_cache(maxsize=None)
def _build_sc_scatter():
    return pl.kernel(
        _sc_scatter_body,
        out_type=jax.ShapeDtypeStruct((NUM_CORES, N_PAD, C_DIM), jnp.float32),
        mesh=plsc.VectorSubcoreMesh(core_axis_name="c", subcore_axis_name="s",
                                    num_cores=NUM_CORES,
                                    num_subcores=NUM_SUBCORES),
        scratch_types=(
            [pltpu.VMEM((CHUNK, BATCH), jnp.int32)] * 2
            + [pltpu.VMEM((BATCH, C_DIM), jnp.float32)] * NBUF
            + [pltpu.VMEM_SHARED((N_PAD, C_DIM), jnp.float32)]
            + [pltpu.SemaphoreType.DMA] * (2 * NBUF)
        ),
    )


def kernel(x, in_idx, out_idx, W1, W2, gamma, beta):
    koffs = (jnp.arange(K_OFF, dtype=jnp.int32) * N_NODES)[:, None]
    in_flat = (in_idx.astype(jnp.int32) + koffs).reshape(-1)
    out_flat = out_idx.astype(jnp.int32).reshape(-1)
    pad = NUM_TILES * EPT - EDGES_TOTAL
    pad_ar = jnp.arange(pad, dtype=jnp.int32)
    in_flat = jnp.concatenate([in_flat, pad_ar % N_NODES])
    out_flat = jnp.concatenate(
        [out_flat, N_NODES + pad_ar % (N_PAD - N_NODES)])
    in3 = in_flat.reshape(NUM_TILES, NCHUNK, CHUNK, BATCH)
    out3 = out_flat.reshape(NUM_TILES, NCHUNK, CHUNK, BATCH)

    gb = jnp.zeros((8, C_DIM), jnp.float32).at[0].set(gamma).at[1].set(beta)

    sc_scatter = _build_sc_scatter()
    y1 = _per_offset_matmul(x, W1)
    p1 = sc_scatter(y1, in3, out3)
    a, stats = _merge_and_stats(p1)
    y2 = _bn_relu_matmul(a, stats, gb, W2)
    p2 = sc_scatter(y2, in3, out3)
    return _residual_relu(p2, x)


# speedup vs baseline: 1.7911x; 1.0197x over previous
import functools

import jax
import jax.numpy as jnp
from jax import lax
from jax.experimental import pallas as pl
from jax.experimental.pallas import tpu as pltpu
from jax.experimental.pallas import tpu_sc as plsc

N_NODES = 10000
C_DIM = 128
K_OFF = 27
E_EDGES = 12000

NUM_CORES = 2
NUM_SUBCORES = 16
NUM_TILES = NUM_CORES * NUM_SUBCORES

BATCH = 128
EDGES_TOTAL = K_OFF * E_EDGES
EPT = 10240
NBATCH = EPT // BATCH
N_PAD = 10240
ROWS_PER_TILE = N_PAD // NUM_SUBCORES

BM = 2000
NB = N_NODES // BM
BM3 = 1024
NBLK3 = N_PAD // BM3
EPS = 1e-5


def _mm_body(x_ref, w_ref, y_ref):
    y_ref[...] = jnp.dot(x_ref[...], w_ref[0],
                         preferred_element_type=jnp.float32)


def _per_offset_matmul(x, w):
    return pl.pallas_call(
        _mm_body,
        grid=(NB, K_OFF),
        in_specs=[
            pl.BlockSpec((BM, C_DIM), lambda i, k: (i, 0)),
            pl.BlockSpec((1, C_DIM, C_DIM), lambda i, k: (k, 0, 0)),
        ],
        out_specs=pl.BlockSpec((BM, C_DIM), lambda i, k: (k * NB + i, 0)),
        out_shape=jax.ShapeDtypeStruct((K_OFF * N_NODES, C_DIM), jnp.float32),
    )(x, w)


def _merge_stats_body(p_ref, a_ref, stats_ref):
    i = pl.program_id(0)
    a = p_ref[0] + p_ref[1]
    a_ref[...] = a
    rows = lax.broadcasted_iota(jnp.int32, (BM3, C_DIM), 0) + i * BM3
    am = jnp.where(rows < N_NODES, a, 0.0)
    s = jnp.sum(am, axis=0, keepdims=True)
    sq = jnp.sum(am * am, axis=0, keepdims=True)
    upd = jnp.concatenate([s, sq, jnp.zeros((6, C_DIM), jnp.float32)], axis=0)

    @pl.when(i == 0)
    def _():
        stats_ref[...] = jnp.zeros((8, C_DIM), jnp.float32)

    stats_ref[...] += upd


def _merge_and_stats(partials):
    return pl.pallas_call(
        _merge_stats_body,
        grid=(NBLK3,),
        in_specs=[pl.BlockSpec((2, BM3, C_DIM), lambda i: (0, i, 0))],
        out_specs=[
            pl.BlockSpec((BM3, C_DIM), lambda i: (i, 0)),
            pl.BlockSpec((8, C_DIM), lambda i: (0, 0)),
        ],
        out_shape=[
            jax.ShapeDtypeStruct((N_PAD, C_DIM), jnp.float32),
            jax.ShapeDtypeStruct((8, C_DIM), jnp.float32),
        ],
    )(partials)


def _bn_mm_body(a_ref, stats_ref, gb_ref, w_ref, y_ref):
    st = stats_ref[...]
    gb = gb_ref[...]
    mean = st[0] * (1.0 / N_NODES)
    var = st[1] * (1.0 / N_NODES) - mean * mean
    inv = lax.rsqrt(var + EPS)
    scale = inv * gb[0]
    shift = gb[1] - mean * scale
    h = jnp.maximum(a_ref[...] * scale + shift, 0.0)
    y_ref[...] = jnp.dot(h, w_ref[0], preferred_element_type=jnp.float32)


def _bn_relu_matmul(a, stats, gb, w):
    return pl.pallas_call(
        _bn_mm_body,
        grid=(NB, K_OFF),
        in_specs=[
            pl.BlockSpec((BM, C_DIM), lambda i, k: (i, 0)),
            pl.BlockSpec((8, C_DIM), lambda i, k: (0, 0)),
            pl.BlockSpec((8, C_DIM), lambda i, k: (0, 0)),
            pl.BlockSpec((1, C_DIM, C_DIM), lambda i, k: (k, 0, 0)),
        ],
        out_specs=pl.BlockSpec((BM, C_DIM), lambda i, k: (k * NB + i, 0)),
        out_shape=jax.ShapeDtypeStruct((K_OFF * N_NODES, C_DIM), jnp.float32),
    )(a, stats, gb, w)


def _final_body(p_ref, x_ref, o_ref):
    o_ref[...] = jnp.maximum(p_ref[0] + p_ref[1] + x_ref[...], 0.0)


def _residual_relu(partials, x):
    return pl.pallas_call(
        _final_body,
        grid=(NB,),
        in_specs=[
            pl.BlockSpec((2, BM, C_DIM), lambda i: (0, i, 0)),
            pl.BlockSpec((BM, C_DIM), lambda i: (i, 0)),
        ],
        out_specs=pl.BlockSpec((BM, C_DIM), lambda i: (i, 0)),
        out_shape=jax.ShapeDtypeStruct((N_NODES, C_DIM), jnp.float32),
    )(partials, x)


NBUF = 2
CHUNK = 16
NCHUNK = NBATCH // CHUNK


def _sc_scatter_body(y_hbm, inidx_hbm, outidx_hbm, out_hbm,
                     in_v0, in_v1, out_v0, out_v1, r0, r1,
                     acc_sh, g0, g1, isem):
    rows = (r0, r1)
    gsem = (g0, g1)
    in_vs = (in_v0, in_v1)
    out_vs = (out_v0, out_v1)
    cid = lax.axis_index("c")
    sid = lax.axis_index("s")
    wid = cid * NUM_SUBCORES + sid

    zero = jnp.zeros((16,), jnp.float32)

    def zrow(r, carry):
        for c8 in range(C_DIM // 16):
            r0[r, pl.ds(c8 * 16, 16)] = zero
        return carry

    lax.fori_loop(0, BATCH, zrow, 0)
    for b in range(ROWS_PER_TILE // BATCH):
        pltpu.sync_copy(r0,
                        acc_sh.at[pl.ds(sid * ROWS_PER_TILE + b * BATCH,
                                        BATCH)])

    plsc.subcore_barrier()

    pltpu.async_copy(inidx_hbm.at[wid, 0], in_vs[0], isem)
    pltpu.async_copy(outidx_hbm.at[wid, 0], out_vs[0], isem)

    for cc in range(NCHUNK):
        p = cc % 2
        in_v = in_vs[p]
        out_v = out_vs[p]
        pltpu.make_async_copy(inidx_hbm.at[wid, cc], in_v, isem).wait()
        pltpu.make_async_copy(outidx_hbm.at[wid, cc], out_v, isem).wait()
        if cc + 1 < NCHUNK:
            pltpu.async_copy(inidx_hbm.at[wid, cc + 1], in_vs[1 - p], isem)
            pltpu.async_copy(outidx_hbm.at[wid, cc + 1], out_vs[1 - p], isem)
        for b in range(NBUF):
            pltpu.async_copy(y_hbm.at[in_v.at[b]], rows[b], gsem[b])

        def body(jj, c2, in_v=in_v, out_v=out_v):
            for b in range(NBUF):
                j = jj * NBUF + b
                pltpu.make_async_copy(y_hbm.at[in_v.at[j]], rows[b],
                                      gsem[b]).wait()
                pltpu.sync_copy(rows[b], acc_sh.at[out_v.at[j]], add=True)

                @pl.when(j + NBUF < CHUNK)
                def _():
                    pltpu.async_copy(y_hbm.at[in_v.at[j + NBUF]], rows[b],
                                     gsem[b])

            return c2

        lax.fori_loop(0, CHUNK // NBUF, body, 0)

    plsc.subcore_barrier()

    pltpu.sync_copy(acc_sh.at[pl.ds(sid * ROWS_PER_TILE, ROWS_PER_TILE)],
                    out_hbm.at[cid, pl.ds(sid * ROWS_PER_TILE,
                                          ROWS_PER_TILE)])


@functools.lru_cache(maxsize=None)
def _build_sc_scatter():
    return pl.kernel(
        _sc_scatter_body,
        out_type=jax.ShapeDtypeStruct((NUM_CORES, N_PAD, C_DIM), jnp.float32),
        mesh=plsc.VectorSubcoreMesh(core_axis_name="c", subcore_axis_name="s",
                                    num_cores=NUM_CORES,
                                    num_subcores=NUM_SUBCORES),
        scratch_types=(
            [pltpu.VMEM((CHUNK, BATCH), jnp.int32)] * 4
            + [pltpu.VMEM((BATCH, C_DIM), jnp.float32)] * NBUF
            + [pltpu.VMEM_SHARED((N_PAD, C_DIM), jnp.float32)]
            + [pltpu.SemaphoreType.DMA] * (NBUF + 1)
        ),
    )


def kernel(x, in_idx, out_idx, W1, W2, gamma, beta):
    koffs = (jnp.arange(K_OFF, dtype=jnp.int32) * N_NODES)[:, None]
    in_flat = (in_idx.astype(jnp.int32) + koffs).reshape(-1)
    out_flat = out_idx.astype(jnp.int32).reshape(-1)
    pad = NUM_TILES * EPT - EDGES_TOTAL
    pad_ar = jnp.arange(pad, dtype=jnp.int32)
    in_flat = jnp.concatenate([in_flat, pad_ar % N_NODES])
    out_flat = jnp.concatenate(
        [out_flat, N_NODES + pad_ar % (N_PAD - N_NODES)])
    in3 = in_flat.reshape(NUM_TILES, NCHUNK, CHUNK, BATCH)
    out3 = out_flat.reshape(NUM_TILES, NCHUNK, CHUNK, BATCH)

    gb = jnp.zeros((8, C_DIM), jnp.float32).at[0].set(gamma).at[1].set(beta)

    sc_scatter = _build_sc_scatter()
    y1 = _per_offset_matmul(x, W1)
    p1 = sc_scatter(y1, in3, out3)
    a, stats = _merge_and_stats(p1)
    y2 = _bn_relu_matmul(a, stats, gb, W2)
    p2 = sc_scatter(y2, in3, out3)
    return _residual_relu(p2, x)


# speedup vs baseline: 1.7935x; 1.0013x over previous
import functools

import jax
import jax.numpy as jnp
from jax import lax
from jax.experimental import pallas as pl
from jax.experimental.pallas import tpu as pltpu
from jax.experimental.pallas import tpu_sc as plsc

N_NODES = 10000
C_DIM = 128
K_OFF = 27
E_EDGES = 12000

NUM_CORES = 2
NUM_SUBCORES = 16
NUM_TILES = NUM_CORES * NUM_SUBCORES

BATCH = 128
EDGES_TOTAL = K_OFF * E_EDGES
EPT = 10240
NBATCH = EPT // BATCH
N_PAD = 10240
ROWS_PER_TILE = N_PAD // NUM_SUBCORES

BM = 2000
NB = N_NODES // BM
BM3 = 1024
NBLK3 = N_PAD // BM3
EPS = 1e-5


def _mm_body(x_ref, w_ref, y_ref):
    y_ref[...] = jnp.dot(x_ref[...], w_ref[0],
                         preferred_element_type=jnp.float32)


def _per_offset_matmul(x, w):
    return pl.pallas_call(
        _mm_body,
        grid=(NB, K_OFF),
        in_specs=[
            pl.BlockSpec((BM, C_DIM), lambda i, k: (i, 0)),
            pl.BlockSpec((1, C_DIM, C_DIM), lambda i, k: (k, 0, 0)),
        ],
        out_specs=pl.BlockSpec((BM, C_DIM), lambda i, k: (k * NB + i, 0)),
        out_shape=jax.ShapeDtypeStruct((K_OFF * N_NODES, C_DIM), jnp.float32),
    )(x, w)


def _merge_stats_body(p_ref, a_ref, stats_ref):
    i = pl.program_id(0)
    a = p_ref[0] + p_ref[1]
    a_ref[...] = a
    rows = lax.broadcasted_iota(jnp.int32, (BM3, C_DIM), 0) + i * BM3
    am = jnp.where(rows < N_NODES, a, 0.0)
    s = jnp.sum(am, axis=0, keepdims=True)
    sq = jnp.sum(am * am, axis=0, keepdims=True)
    upd = jnp.concatenate([s, sq, jnp.zeros((6, C_DIM), jnp.float32)], axis=0)

    @pl.when(i == 0)
    def _():
        stats_ref[...] = jnp.zeros((8, C_DIM), jnp.float32)

    stats_ref[...] += upd


def _merge_and_stats(partials):
    return pl.pallas_call(
        _merge_stats_body,
        grid=(NBLK3,),
        in_specs=[pl.BlockSpec((2, BM3, C_DIM), lambda i: (0, i, 0))],
        out_specs=[
            pl.BlockSpec((BM3, C_DIM), lambda i: (i, 0)),
            pl.BlockSpec((8, C_DIM), lambda i: (0, 0)),
        ],
        out_shape=[
            jax.ShapeDtypeStruct((N_PAD, C_DIM), jnp.float32),
            jax.ShapeDtypeStruct((8, C_DIM), jnp.float32),
        ],
    )(partials)


def _bn_mm_body(a_ref, stats_ref, gb_ref, w_ref, y_ref):
    st = stats_ref[...]
    gb = gb_ref[...]
    mean = st[0] * (1.0 / N_NODES)
    var = st[1] * (1.0 / N_NODES) - mean * mean
    inv = lax.rsqrt(var + EPS)
    scale = inv * gb[0]
    shift = gb[1] - mean * scale
    h = jnp.maximum(a_ref[...] * scale + shift, 0.0)
    y_ref[...] = jnp.dot(h, w_ref[0], preferred_element_type=jnp.float32)


def _bn_relu_matmul(a, stats, gb, w):
    return pl.pallas_call(
        _bn_mm_body,
        grid=(NB, K_OFF),
        in_specs=[
            pl.BlockSpec((BM, C_DIM), lambda i, k: (i, 0)),
            pl.BlockSpec((8, C_DIM), lambda i, k: (0, 0)),
            pl.BlockSpec((8, C_DIM), lambda i, k: (0, 0)),
            pl.BlockSpec((1, C_DIM, C_DIM), lambda i, k: (k, 0, 0)),
        ],
        out_specs=pl.BlockSpec((BM, C_DIM), lambda i, k: (k * NB + i, 0)),
        out_shape=jax.ShapeDtypeStruct((K_OFF * N_NODES, C_DIM), jnp.float32),
    )(a, stats, gb, w)


def _final_body(p_ref, x_ref, o_ref):
    o_ref[...] = jnp.maximum(p_ref[0] + p_ref[1] + x_ref[...], 0.0)


def _residual_relu(partials, x):
    return pl.pallas_call(
        _final_body,
        grid=(NB,),
        in_specs=[
            pl.BlockSpec((2, BM, C_DIM), lambda i: (0, i, 0)),
            pl.BlockSpec((BM, C_DIM), lambda i: (i, 0)),
        ],
        out_specs=pl.BlockSpec((BM, C_DIM), lambda i: (i, 0)),
        out_shape=jax.ShapeDtypeStruct((N_NODES, C_DIM), jnp.float32),
    )(partials, x)


NBUF = 2
CHUNK = 16
NCHUNK = NBATCH // CHUNK


def _sc_scatter_body(y_hbm, inidx_hbm, outidx_hbm, out_hbm,
                     in_v0, in_v1, out_v0, out_v1, r0, r1,
                     acc_sh, g0, g1, isem, osem):
    rows = (r0, r1)
    gsem = (g0, g1)
    in_vs = (in_v0, in_v1)
    out_vs = (out_v0, out_v1)
    cid = lax.axis_index("c")
    sid = lax.axis_index("s")
    wid = cid * NUM_SUBCORES + sid

    zero = jnp.zeros((16,), jnp.float32)

    def zrow(r, carry):
        for c8 in range(C_DIM // 16):
            r0[r, pl.ds(c8 * 16, 16)] = zero
        return carry

    lax.fori_loop(0, BATCH, zrow, 0)
    for b in range(ROWS_PER_TILE // BATCH):
        pltpu.sync_copy(r0,
                        acc_sh.at[pl.ds(sid * ROWS_PER_TILE + b * BATCH,
                                        BATCH)])

    plsc.subcore_barrier()

    pltpu.async_copy(inidx_hbm.at[wid, 0], in_vs[0], isem)
    pltpu.async_copy(outidx_hbm.at[wid, 0], out_vs[0], osem)

    for cc in range(NCHUNK):
        p = cc % 2
        in_v = in_vs[p]
        out_v = out_vs[p]
        pltpu.make_async_copy(inidx_hbm.at[wid, cc], in_v, isem).wait()
        pltpu.make_async_copy(outidx_hbm.at[wid, cc], out_v, osem).wait()
        if cc + 1 < NCHUNK:
            pltpu.async_copy(inidx_hbm.at[wid, cc + 1], in_vs[1 - p], isem)
            pltpu.async_copy(outidx_hbm.at[wid, cc + 1], out_vs[1 - p], osem)
        for b in range(NBUF):
            pltpu.async_copy(y_hbm.at[in_v.at[b]], rows[b], gsem[b])

        def body(jj, c2, in_v=in_v, out_v=out_v):
            for b in range(NBUF):
                j = jj * NBUF + b
                pltpu.make_async_copy(y_hbm.at[in_v.at[j]], rows[b],
                                      gsem[b]).wait()
                pltpu.sync_copy(rows[b], acc_sh.at[out_v.at[j]], add=True)

                @pl.when(j + NBUF < CHUNK)
                def _():
                    pltpu.async_copy(y_hbm.at[in_v.at[j + NBUF]], rows[b],
                                     gsem[b])

            return c2

        lax.fori_loop(0, CHUNK // NBUF, body, 0)

    plsc.subcore_barrier()

    pltpu.sync_copy(acc_sh.at[pl.ds(sid * ROWS_PER_TILE, ROWS_PER_TILE)],
                    out_hbm.at[cid, pl.ds(sid * ROWS_PER_TILE,
                                          ROWS_PER_TILE)])


@functools.lru_cache(maxsize=None)
def _build_sc_scatter():
    return pl.kernel(
        _sc_scatter_body,
        out_type=jax.ShapeDtypeStruct((NUM_CORES, N_PAD, C_DIM), jnp.float32),
        mesh=plsc.VectorSubcoreMesh(core_axis_name="c", subcore_axis_name="s",
                                    num_cores=NUM_CORES,
                                    num_subcores=NUM_SUBCORES),
        scratch_types=(
            [pltpu.VMEM((CHUNK, BATCH), jnp.int32)] * 4
            + [pltpu.VMEM((BATCH, C_DIM), jnp.float32)] * NBUF
            + [pltpu.VMEM_SHARED((N_PAD, C_DIM), jnp.float32)]
            + [pltpu.SemaphoreType.DMA] * (NBUF + 2)
        ),
    )


def kernel(x, in_idx, out_idx, W1, W2, gamma, beta):
    koffs = (jnp.arange(K_OFF, dtype=jnp.int32) * N_NODES)[:, None]
    in_flat = (in_idx.astype(jnp.int32) + koffs).reshape(-1)
    out_flat = out_idx.astype(jnp.int32).reshape(-1)
    pad = NUM_TILES * EPT - EDGES_TOTAL
    pad_ar = jnp.arange(pad, dtype=jnp.int32)
    in_flat = jnp.concatenate([in_flat, pad_ar % N_NODES])
    out_flat = jnp.concatenate(
        [out_flat, N_NODES + pad_ar % (N_PAD - N_NODES)])
    in3 = in_flat.reshape(NUM_TILES, NCHUNK, CHUNK, BATCH)
    out3 = out_flat.reshape(NUM_TILES, NCHUNK, CHUNK, BATCH)

    gb = jnp.zeros((8, C_DIM), jnp.float32).at[0].set(gamma).at[1].set(beta)

    sc_scatter = _build_sc_scatter()
    y1 = _per_offset_matmul(x, W1)
    p1 = sc_scatter(y1, in3, out3)
    a, stats = _merge_and_stats(p1)
    y2 = _bn_relu_matmul(a, stats, gb, W2)
    p2 = sc_scatter(y2, in3, out3)
    return _residual_relu(p2, x)


# speedup vs baseline: 1.9578x; 1.0916x over previous
import functools

import jax
import jax.numpy as jnp
from jax import lax
from jax.experimental import pallas as pl
from jax.experimental.pallas import tpu as pltpu
from jax.experimental.pallas import tpu_sc as plsc

N_NODES = 10000
C_DIM = 128
K_OFF = 27
KS = 14
E_EDGES = 12000

NUM_CORES = 2
NUM_SUBCORES = 16
NUM_TILES = NUM_CORES * NUM_SUBCORES

BATCH = 128
EDGES_TOTAL = K_OFF * E_EDGES
EDGES_HALF = EDGES_TOTAL // 2
EPT = 5120
NBATCH = EPT // BATCH
N_PAD = 10240
ROWS_PER_TILE = N_PAD // NUM_SUBCORES

BM = 2000
NB = N_NODES // BM
BM3 = 1024
NBLK3 = N_PAD // BM3
EPS = 1e-5


def _mm_body(x_ref, w_ref, y_ref):
    y_ref[...] = jnp.dot(x_ref[...], w_ref[0],
                         preferred_element_type=jnp.float32)


def _per_offset_matmul(x, w):
    return pl.pallas_call(
        _mm_body,
        grid=(NB, KS),
        in_specs=[
            pl.BlockSpec((BM, C_DIM), lambda i, k: (i, 0)),
            pl.BlockSpec((1, C_DIM, C_DIM), lambda i, k: (k, 0, 0)),
        ],
        out_specs=pl.BlockSpec((BM, C_DIM), lambda i, k: (k * NB + i, 0)),
        out_shape=jax.ShapeDtypeStruct((KS * N_NODES, C_DIM), jnp.float32),
    )(x, w)


def _merge_stats_body(plo_ref, phi_ref, a_ref, stats_ref):
    i = pl.program_id(0)
    a = (plo_ref[0] + plo_ref[1]) + (phi_ref[0] + phi_ref[1])
    a_ref[...] = a
    rows = lax.broadcasted_iota(jnp.int32, (BM3, C_DIM), 0) + i * BM3
    am = jnp.where(rows < N_NODES, a, 0.0)
    s = jnp.sum(am, axis=0, keepdims=True)
    sq = jnp.sum(am * am, axis=0, keepdims=True)
    upd = jnp.concatenate([s, sq, jnp.zeros((6, C_DIM), jnp.float32)], axis=0)

    @pl.when(i == 0)
    def _():
        stats_ref[...] = jnp.zeros((8, C_DIM), jnp.float32)

    stats_ref[...] += upd


def _merge_and_stats(plo, phi):
    return pl.pallas_call(
        _merge_stats_body,
        grid=(NBLK3,),
        in_specs=[
            pl.BlockSpec((2, BM3, C_DIM), lambda i: (0, i, 0)),
            pl.BlockSpec((2, BM3, C_DIM), lambda i: (0, i, 0)),
        ],
        out_specs=[
            pl.BlockSpec((BM3, C_DIM), lambda i: (i, 0)),
            pl.BlockSpec((8, C_DIM), lambda i: (0, 0)),
        ],
        out_shape=[
            jax.ShapeDtypeStruct((N_PAD, C_DIM), jnp.float32),
            jax.ShapeDtypeStruct((8, C_DIM), jnp.float32),
        ],
    )(plo, phi)


def _bn_mm_body(a_ref, stats_ref, gb_ref, w_ref, y_ref):
    st = stats_ref[...]
    gb = gb_ref[...]
    mean = st[0] * (1.0 / N_NODES)
    var = st[1] * (1.0 / N_NODES) - mean * mean
    inv = lax.rsqrt(var + EPS)
    scale = inv * gb[0]
    shift = gb[1] - mean * scale
    h = jnp.maximum(a_ref[...] * scale + shift, 0.0)
    y_ref[...] = jnp.dot(h, w_ref[0], preferred_element_type=jnp.float32)


def _bn_relu_matmul(a, stats, gb, w):
    return pl.pallas_call(
        _bn_mm_body,
        grid=(NB, KS),
        in_specs=[
            pl.BlockSpec((BM, C_DIM), lambda i, k: (i, 0)),
            pl.BlockSpec((8, C_DIM), lambda i, k: (0, 0)),
            pl.BlockSpec((8, C_DIM), lambda i, k: (0, 0)),
            pl.BlockSpec((1, C_DIM, C_DIM), lambda i, k: (k, 0, 0)),
        ],
        out_specs=pl.BlockSpec((BM, C_DIM), lambda i, k: (k * NB + i, 0)),
        out_shape=jax.ShapeDtypeStruct((KS * N_NODES, C_DIM), jnp.float32),
    )(a, stats, gb, w)


def _final_body(plo_ref, phi_ref, x_ref, o_ref):
    a = (plo_ref[0] + plo_ref[1]) + (phi_ref[0] + phi_ref[1])
    o_ref[...] = jnp.maximum(a + x_ref[...], 0.0)


def _residual_relu(plo, phi, x):
    return pl.pallas_call(
        _final_body,
        grid=(NB,),
        in_specs=[
            pl.BlockSpec((2, BM, C_DIM), lambda i: (0, i, 0)),
            pl.BlockSpec((2, BM, C_DIM), lambda i: (0, i, 0)),
            pl.BlockSpec((BM, C_DIM), lambda i: (i, 0)),
        ],
        out_specs=pl.BlockSpec((BM, C_DIM), lambda i: (i, 0)),
        out_shape=jax.ShapeDtypeStruct((N_NODES, C_DIM), jnp.float32),
    )(plo, phi, x)


NBUF = 2
CHUNK = 8
NCHUNK = NBATCH // CHUNK


def _sc_scatter_body(y_hbm, inidx_hbm, outidx_hbm, out_hbm,
                     in_v0, in_v1, out_v0, out_v1, r0, r1,
                     acc_sh, g0, g1, isem, osem):
    rows = (r0, r1)
    gsem = (g0, g1)
    in_vs = (in_v0, in_v1)
    out_vs = (out_v0, out_v1)
    cid = lax.axis_index("c")
    sid = lax.axis_index("s")
    wid = cid * NUM_SUBCORES + sid

    zero = jnp.zeros((16,), jnp.float32)

    def zrow(r, carry):
        for c16 in range(C_DIM // 16):
            r0[r, pl.ds(c16 * 16, 16)] = zero
        return carry

    lax.fori_loop(0, BATCH, zrow, 0)
    for b in range(ROWS_PER_TILE // BATCH):
        pltpu.sync_copy(r0,
                        acc_sh.at[pl.ds(sid * ROWS_PER_TILE + b * BATCH,
                                        BATCH)])

    plsc.subcore_barrier()

    pltpu.async_copy(inidx_hbm.at[wid, 0], in_vs[0], isem)
    pltpu.async_copy(outidx_hbm.at[wid, 0], out_vs[0], osem)

    for cc in range(NCHUNK):
        p = cc % 2
        in_v = in_vs[p]
        out_v = out_vs[p]
        pltpu.make_async_copy(inidx_hbm.at[wid, cc], in_v, isem).wait()
        pltpu.make_async_copy(outidx_hbm.at[wid, cc], out_v, osem).wait()
        if cc + 1 < NCHUNK:
            pltpu.async_copy(inidx_hbm.at[wid, cc + 1], in_vs[1 - p], isem)
            pltpu.async_copy(outidx_hbm.at[wid, cc + 1], out_vs[1 - p], osem)
        for b in range(NBUF):
            pltpu.async_copy(y_hbm.at[in_v.at[b]], rows[b], gsem[b])

        def body(jj, c2, in_v=in_v, out_v=out_v):
            for b in range(NBUF):
                j = jj * NBUF + b
                pltpu.make_async_copy(y_hbm.at[in_v.at[j]], rows[b],
                                      gsem[b]).wait()
                pltpu.sync_copy(rows[b], acc_sh.at[out_v.at[j]], add=True)

                @pl.when(j + NBUF < CHUNK)
                def _():
                    pltpu.async_copy(y_hbm.at[in_v.at[j + NBUF]], rows[b],
                                     gsem[b])

            return c2

        lax.fori_loop(0, CHUNK // NBUF, body, 0)

    plsc.subcore_barrier()

    pltpu.sync_copy(acc_sh.at[pl.ds(sid * ROWS_PER_TILE, ROWS_PER_TILE)],
                    out_hbm.at[cid, pl.ds(sid * ROWS_PER_TILE,
                                          ROWS_PER_TILE)])


@functools.lru_cache(maxsize=None)
def _build_sc_scatter():
    return pl.kernel(
        _sc_scatter_body,
        out_type=jax.ShapeDtypeStruct((NUM_CORES, N_PAD, C_DIM),
                                      jnp.float32),
        mesh=plsc.VectorSubcoreMesh(core_axis_name="c", subcore_axis_name="s",
                                    num_cores=NUM_CORES,
                                    num_subcores=NUM_SUBCORES),
        scratch_types=(
            [pltpu.VMEM((CHUNK, BATCH), jnp.int32)] * 4
            + [pltpu.VMEM((BATCH, C_DIM), jnp.float32)] * NBUF
            + [pltpu.VMEM_SHARED((N_PAD, C_DIM), jnp.float32)]
            + [pltpu.SemaphoreType.DMA] * (NBUF + 2)
        ),
    )


def _edge_arrays(flat, scrap_mod):
    pad = NUM_TILES * EPT - EDGES_HALF
    pad_ar = jnp.arange(pad, dtype=jnp.int32)
    flat = jnp.concatenate([flat, scrap_mod(pad_ar)])
    return flat.reshape(NUM_TILES, NCHUNK, CHUNK, BATCH)


def kernel(x, in_idx, out_idx, W1, W2, gamma, beta):
    koffs = (jnp.arange(K_OFF, dtype=jnp.int32) * N_NODES)[:, None]
    in_flat = (in_idx.astype(jnp.int32) + koffs).reshape(-1)
    out_flat = out_idx.astype(jnp.int32).reshape(-1)
    inA = _edge_arrays(in_flat[:EDGES_HALF], lambda p: p % N_NODES)
    inB = _edge_arrays(in_flat[EDGES_HALF:] - 13 * N_NODES,
                       lambda p: p % N_NODES)
    scrap = lambda p: N_NODES + p % (N_PAD - N_NODES)
    outA = _edge_arrays(out_flat[:EDGES_HALF], scrap)
    outB = _edge_arrays(out_flat[EDGES_HALF:], scrap)

    gb = jnp.zeros((8, C_DIM), jnp.float32).at[0].set(gamma).at[1].set(beta)
    w1a, w1b = W1[:KS], W1[K_OFF - KS:]
    w2a, w2b = W2[:KS], W2[K_OFF - KS:]

    sc_scatter = _build_sc_scatter()

    y1a = _per_offset_matmul(x, w1a)
    p1a = sc_scatter(y1a, inA, outA)
    y1b = _per_offset_matmul(x, w1b)
    p1b = sc_scatter(y1b, inB, outB)
    a, stats = _merge_and_stats(p1a, p1b)
    y2a = _bn_relu_matmul(a, stats, gb, w2a)
    p2a = sc_scatter(y2a, inA, outA)
    y2b = _bn_relu_matmul(a, stats, gb, w2b)
    p2b = sc_scatter(y2b, inB, outB)
    return _residual_relu(p2a, p2b, x)
